# Initial kernel scaffold; baseline (speedup 1.0000x reference)
import jax
import jax.numpy as jnp
from jax.experimental import pallas as pl


def kernel(x, edge_index, edge_attr, batch, lin1_w, lin1_b, eps1, W1a, b1a, W1b, b1b, lin2_w, lin2_b, eps2, W2a, b2a, W2b, b2b, Wf1, bf1, Wf2, bf2):
    raise NotImplementedError("write your pallas kernel here")


# speedup vs baseline: 2.3531x; 2.3531x over previous
import functools

import jax
import jax.numpy as jnp
from jax import lax
from jax.experimental import pallas as pl
from jax.experimental.pallas import tpu as pltpu
from jax.experimental.pallas import tpu_sc as plsc

_N = 10000
_E = 320000
_D = 128
_NG = 64

_NSC = 2
_NTILE = 16
_NW = _NSC * _NTILE
_C = 80
_EPW = _E // _NW
_NCHUNK = _EPW // _C
_ROWCHUNKS = _N // _C
_RCPT = -(-_ROWCHUNKS // _NTILE)


def _sc_message_body(x_hbm, src_hbm, dst_hbm, e_hbm, out_hbm,
                     idx_s, idx_d, xbuf, ebuf, aggr_sh, sem):
    c = lax.axis_index("c")
    s = lax.axis_index("s")
    wid = c * _NTILE + s
    ebase = wid * _EPW

    def zrow(i, carry):
        xbuf[i // 8, pl.ds((i % 8) * 16, 16)] = jnp.zeros((16,), jnp.float32)
        return carry
    lax.fori_loop(0, _C * (_D // 16), zrow, 0)
    for j in range(_RCPT):
        idx = s + _NTILE * j

        @pl.when(idx < _ROWCHUNKS)
        def _():
            pltpu.sync_copy(xbuf, aggr_sh.at[pl.ds(idx * _C, _C)])
    plsc.subcore_barrier()

    def chunk(k, carry):
        base = ebase + k * _C
        pltpu.sync_copy(src_hbm.at[pl.ds(base, _C)], idx_s)
        pltpu.sync_copy(dst_hbm.at[pl.ds(base, _C)], idx_d)
        pltpu.sync_copy(e_hbm.at[pl.ds(base, _C)], ebuf)
        pltpu.async_copy(x_hbm.at[idx_s], xbuf, sem).wait()

        def row(r, rc):
            for g in range(_D // 16):
                sl = pl.ds(g * 16, 16)
                ebuf[r, sl] = jnp.maximum(xbuf[r, sl] + ebuf[r, sl], 0.0)
            return rc
        lax.fori_loop(0, _C, row, 0)

        pltpu.sync_copy(ebuf, aggr_sh.at[idx_d], add=True)
        return carry
    lax.fori_loop(0, _NCHUNK, chunk, 0)
    plsc.subcore_barrier()

    for j in range(_RCPT):
        idx = s + _NTILE * j

        @pl.when(idx < _ROWCHUNKS)
        def _():
            pltpu.sync_copy(aggr_sh.at[pl.ds(idx * _C, _C)], xbuf)
            pltpu.sync_copy(xbuf, out_hbm.at[pl.ds(c * _N + idx * _C, _C)])


def _sc_message(x, src, dst, e):
    mesh = plsc.VectorSubcoreMesh(core_axis_name="c", subcore_axis_name="s")
    f = pl.kernel(
        _sc_message_body,
        out_type=jax.ShapeDtypeStruct((_NSC * _N, _D), jnp.float32),
        mesh=mesh,
        scratch_types=[
            pltpu.VMEM((_C,), jnp.int32),
            pltpu.VMEM((_C,), jnp.int32),
            pltpu.VMEM((_C, _D), jnp.float32),
            pltpu.VMEM((_C, _D), jnp.float32),
            pltpu.VMEM_SHARED((_N, _D), jnp.float32),
            pltpu.SemaphoreType.DMA,
        ],
    )
    return f(x, src, dst, e)


def _edge_mlp(edge_attr, w12, b12):
    be = 1280

    def body(a_ref, w_ref, b_ref, o1_ref, o2_ref):
        r = jnp.dot(a_ref[...], w_ref[...],
                    preferred_element_type=jnp.float32) + b_ref[...]
        o1_ref[...] = r[:, :_D]
        o2_ref[...] = r[:, _D:]

    return pl.pallas_call(
        body,
        grid=(_E // be,),
        in_specs=[
            pl.BlockSpec((be, 16), lambda i: (i, 0)),
            pl.BlockSpec((16, 2 * _D), lambda i: (0, 0)),
            pl.BlockSpec((1, 2 * _D), lambda i: (0, 0)),
        ],
        out_specs=[pl.BlockSpec((be, _D), lambda i: (i, 0)),
                   pl.BlockSpec((be, _D), lambda i: (i, 0))],
        out_shape=[jax.ShapeDtypeStruct((_E, _D), jnp.float32),
                   jax.ShapeDtypeStruct((_E, _D), jnp.float32)],
    )(edge_attr, w12, b12)


def _node_mlp(x, aggr2n, eps, wa, ba, wb, bb):
    br = 1000
    nblk = _N // br

    def body(x_ref, a0_ref, a1_ref, eps_ref, wa_ref, ba_ref, wb_ref, bb_ref,
             o_ref):
        h = (1.0 + eps_ref[0]) * x_ref[...] + a0_ref[...] + a1_ref[...]
        t = jnp.maximum(
            jnp.dot(h, wa_ref[...], preferred_element_type=jnp.float32)
            + ba_ref[...], 0.0)
        o_ref[...] = jnp.maximum(
            jnp.dot(t, wb_ref[...], preferred_element_type=jnp.float32)
            + bb_ref[...], 0.0)

    return pl.pallas_call(
        body,
        grid=(nblk,),
        in_specs=[
            pl.BlockSpec((br, _D), lambda i: (i, 0)),
            pl.BlockSpec((br, _D), lambda i: (i, 0)),
            pl.BlockSpec((br, _D), lambda i: (i + nblk, 0)),
            pl.BlockSpec(memory_space=pltpu.SMEM),
            pl.BlockSpec((_D, _D), lambda i: (0, 0)),
            pl.BlockSpec((1, _D), lambda i: (0, 0)),
            pl.BlockSpec((_D, _D), lambda i: (0, 0)),
            pl.BlockSpec((1, _D), lambda i: (0, 0)),
        ],
        out_specs=pl.BlockSpec((br, _D), lambda i: (i, 0)),
        out_shape=jax.ShapeDtypeStruct((_N, _D), jnp.float32),
    )(x, aggr2n, aggr2n, eps, wa, ba, wb, bb)


def _pool_fc(h, batch2d, wf1, bf1, wf2p, bf2p):
    def body(h_ref, b_ref, w1_ref, b1_ref, w2_ref, b2_ref, o_ref):
        ids = lax.broadcasted_iota(jnp.int32, (_NG, _N), 0)
        onehot = jnp.where(ids == b_ref[...], 1.0, 0.0)
        pooled = jnp.dot(onehot, h_ref[...], preferred_element_type=jnp.float32)
        t = jnp.maximum(
            jnp.dot(pooled, w1_ref[...], preferred_element_type=jnp.float32)
            + b1_ref[...], 0.0)
        o_ref[...] = jnp.dot(t, w2_ref[...],
                             preferred_element_type=jnp.float32) + b2_ref[...]

    return pl.pallas_call(
        body,
        out_shape=jax.ShapeDtypeStruct((_NG, _D), jnp.float32),
    )(h, batch2d, wf1, bf1, wf2p, bf2p)


def kernel(x, edge_index, edge_attr, batch,
           lin1_w, lin1_b, eps1, W1a, b1a, W1b, b1b,
           lin2_w, lin2_b, eps2, W2a, b2a, W2b, b2b,
           Wf1, bf1, Wf2, bf2):
    src = edge_index[0]
    dst = edge_index[1]

    w12 = jnp.concatenate([lin1_w, lin2_w], axis=1)
    b12 = jnp.concatenate([lin1_b, lin2_b])[None, :]
    e1, e2 = _edge_mlp(edge_attr, w12, b12)

    aggr1 = _sc_message(x, src, dst, e1)
    h1 = _node_mlp(x, aggr1, eps1[None], W1a, b1a[None, :], W1b, b1b[None, :])

    aggr2 = _sc_message(h1, src, dst, e2)
    h2 = _node_mlp(h1, aggr2, eps2[None], W2a, b2a[None, :], W2b, b2b[None, :])

    wf2p = jnp.pad(Wf2, ((0, 0), (0, _D - Wf2.shape[1])))
    bf2p = jnp.pad(bf2, (0, _D - bf2.shape[0]))[None, :]
    out = _pool_fc(h2, batch[None, :], Wf1, bf1[None, :], wf2p, bf2p)
    return out[:, :Wf2.shape[1]]
